# Initial kernel scaffold; baseline (speedup 1.0000x reference)
import jax
import jax.numpy as jnp
from jax.experimental import pallas as pl


def kernel(class_logits, box_regression, proposals):
    raise NotImplementedError("write your pallas kernel here")


# speedup vs baseline: 1.1445x; 1.1445x over previous
import jax
import jax.numpy as jnp
import numpy as np
from jax.experimental import pallas as pl
from jax.experimental.pallas import tpu as pltpu

_N = 5000
_C = 91
_SCORE_THRESH = 0.05
_NMS_THRESH = 0.5
_DET = 100
_K = 1000
_KP = 1024
_IMG = 800.0
_OFF = _IMG + 1.0
_CLIP = float(np.log(1000.0 / 16.0))


def _scores_body(logits_ref, out_ref):
    x = logits_ref[...]
    m = jnp.max(x, axis=-1, keepdims=True)
    e = jnp.exp(x - m)
    p = e / jnp.sum(e, axis=-1, keepdims=True)
    cls = jax.lax.broadcasted_iota(jnp.int32, x.shape, 1)
    valid = (cls >= 1) & (p > _SCORE_THRESH)
    out_ref[...] = jnp.where(valid, p, -1.0)


def _decode(rx, ry, rw, rh, px1, py1, px2, py2):
    w = px2 - px1
    h = py2 - py1
    cx = px1 + 0.5 * w
    cy = py1 + 0.5 * h
    dx = rx / 10.0
    dy = ry / 10.0
    dw = jnp.minimum(rw / 5.0, _CLIP)
    dh = jnp.minimum(rh / 5.0, _CLIP)
    pcx = dx * w + cx
    pcy = dy * h + cy
    pw = jnp.exp(dw) * w
    ph = jnp.exp(dh) * h
    x1 = jnp.clip(pcx - 0.5 * pw, 0.0, _IMG)
    y1 = jnp.clip(pcy - 0.5 * ph, 0.0, _IMG)
    x2 = jnp.clip(pcx + 0.5 * pw, 0.0, _IMG)
    y2 = jnp.clip(pcy + 0.5 * ph, 0.0, _IMG)
    return x1, y1, x2, y2


def _nms_body(rel_ref, prop_ref, relT_ref, propT_ref, clsc_ref, clsr_ref,
              sc_ref, sr_ref, out_ref, scr_ref, iou_ref, sw_ref):
    x1c, y1c, x2c, y2c = _decode(
        rel_ref[:, 0:1], rel_ref[:, 1:2], rel_ref[:, 2:3], rel_ref[:, 3:4],
        prop_ref[:, 0:1], prop_ref[:, 1:2], prop_ref[:, 2:3], prop_ref[:, 3:4])
    x1r, y1r, x2r, y2r = _decode(
        relT_ref[0:1, :], relT_ref[1:2, :], relT_ref[2:3, :], relT_ref[3:4, :],
        propT_ref[0:1, :], propT_ref[1:2, :], propT_ref[2:3, :], propT_ref[3:4, :])
    offc = clsc_ref[...] * _OFF
    offr = clsr_ref[...] * _OFF
    ax1, ay1, ax2, ay2 = x1c + offc, y1c + offc, x2c + offc, y2c + offc
    bx1, by1, bx2, by2 = x1r + offr, y1r + offr, x2r + offr, y2r + offr
    areac = (ax2 - ax1) * (ay2 - ay1)
    arear = (bx2 - bx1) * (by2 - by1)
    iw = jnp.maximum(jnp.minimum(ax2, bx2) - jnp.maximum(ax1, bx1), 0.0)
    ih = jnp.maximum(jnp.minimum(ay2, by2) - jnp.maximum(ay1, by1), 0.0)
    inter = iw * ih
    iou_ref[...] = inter / (areac + arear - inter + 1e-9)
    scr_ref[:, 0:1] = x1c
    scr_ref[:, 1:2] = y1c
    scr_ref[:, 2:3] = x2c
    scr_ref[:, 3:4] = y2c
    scr_ref[:, 4:5] = sc_ref[...]
    scr_ref[:, 5:8] = jnp.zeros((_KP, 3), jnp.float32)
    sw_ref[...] = sr_ref[...]
    lane = jax.lax.broadcasted_iota(jnp.int32, (1, _KP), 1)

    def body(i, carry):
        sw = sw_ref[...]
        m = jnp.max(sw)
        j = jnp.min(jnp.where(sw == m, lane, _KP))
        out_ref[pl.ds(i, 1), 0:5] = scr_ref[pl.ds(j, 1), 0:5]
        row = iou_ref[pl.ds(j, 1), :]
        sw_ref[...] = jnp.where(row > _NMS_THRESH, -jnp.inf, sw)
        return carry

    jax.lax.fori_loop(0, _DET, body, 0)


@jax.jit
def kernel(class_logits, box_regression, proposals):
    scores_m = pl.pallas_call(
        _scores_body,
        out_shape=jax.ShapeDtypeStruct((_N, _C), jnp.float32),
    )(class_logits)

    top_scores, top_idx = jax.lax.top_k(scores_m.reshape(-1), _K)
    rows = top_idx // _C
    cls_f = (top_idx % _C).astype(jnp.float32)
    rel = box_regression.reshape(_N * _C, 4)[top_idx]
    prop = proposals[rows]

    pad = _KP - _K
    relp = jnp.concatenate([rel, jnp.zeros((pad, 4), jnp.float32)], axis=0)
    propp = jnp.concatenate([prop, jnp.zeros((pad, 4), jnp.float32)], axis=0)
    clsp = jnp.concatenate([cls_f, jnp.zeros((pad,), jnp.float32)], axis=0)
    scp = jnp.concatenate(
        [top_scores, jnp.full((pad,), -jnp.inf, jnp.float32)], axis=0)

    out = pl.pallas_call(
        _nms_body,
        out_shape=jax.ShapeDtypeStruct((_DET, 8), jnp.float32),
        scratch_shapes=[
            pltpu.VMEM((_KP, 8), jnp.float32),
            pltpu.VMEM((_KP, _KP), jnp.float32),
            pltpu.VMEM((1, _KP), jnp.float32),
        ],
    )(relp, propp, relp.T, propp.T,
      clsp[:, None], clsp[None, :], scp[:, None], scp[None, :])
    return out[:, :5]


# speedup vs baseline: 1.4529x; 1.2694x over previous
import jax
import jax.numpy as jnp
import numpy as np
from jax.experimental import pallas as pl
from jax.experimental.pallas import tpu as pltpu

_N = 5000
_C = 91
_SCORE_THRESH = 0.05
_NMS_THRESH = 0.5
_DET = 100
_K = 1000
_KP = 1024
_IMG = 800.0
_OFF = _IMG + 1.0
_CLIP = float(np.log(1000.0 / 16.0))


def _scores_body(logits_ref, out_ref):
    x = logits_ref[...]
    m = jnp.max(x, axis=-1, keepdims=True)
    e = jnp.exp(x - m)
    p = e / jnp.sum(e, axis=-1, keepdims=True)
    cls = jax.lax.broadcasted_iota(jnp.int32, x.shape, 1)
    valid = (cls >= 1) & (p > _SCORE_THRESH)
    out_ref[...] = jnp.where(valid, p, -1.0)


def _decode(rx, ry, rw, rh, px1, py1, px2, py2):
    w = px2 - px1
    h = py2 - py1
    cx = px1 + 0.5 * w
    cy = py1 + 0.5 * h
    dx = rx / 10.0
    dy = ry / 10.0
    dw = jnp.minimum(rw / 5.0, _CLIP)
    dh = jnp.minimum(rh / 5.0, _CLIP)
    pcx = dx * w + cx
    pcy = dy * h + cy
    pw = jnp.exp(dw) * w
    ph = jnp.exp(dh) * h
    x1 = jnp.clip(pcx - 0.5 * pw, 0.0, _IMG)
    y1 = jnp.clip(pcy - 0.5 * ph, 0.0, _IMG)
    x2 = jnp.clip(pcx + 0.5 * pw, 0.0, _IMG)
    y2 = jnp.clip(pcy + 0.5 * ph, 0.0, _IMG)
    return x1, y1, x2, y2


def _nms_body(bp_ref, rid_ref, icls_ref, sc_ref,
              out_ref, scr_ref, iou_ref, sw_ref):
    rid = rid_ref[...]
    dn = (((1,), (0,)), ((), ()))
    nb = 5
    bs = _N // nb

    def gather_step(b, acc):
        li = jax.lax.broadcasted_iota(jnp.int32, (_KP, bs), 1) + b * bs
        ohb = (li == rid).astype(jnp.float32)
        blk = bp_ref[pl.ds(pl.multiple_of(b * bs, 8), bs), :]
        return acc + jax.lax.dot_general(
            ohb, blk, dn,
            precision=jax.lax.Precision.HIGHEST,
            preferred_element_type=jnp.float32)

    AP = jax.lax.fori_loop(
        0, nb, gather_step, jnp.zeros((_KP, 4 * _C + 4), jnp.float32))
    A = AP[:, 0:4 * _C]
    P = AP[:, 4 * _C:4 * _C + 4]
    icls = icls_ref[...]
    lane = jax.lax.broadcasted_iota(jnp.int32, (_KP, 4 * _C), 1)
    base = icls * 4
    rx = jnp.sum(jnp.where(lane == base, A, 0.0), axis=1, keepdims=True)
    ry = jnp.sum(jnp.where(lane == base + 1, A, 0.0), axis=1, keepdims=True)
    rw = jnp.sum(jnp.where(lane == base + 2, A, 0.0), axis=1, keepdims=True)
    rh = jnp.sum(jnp.where(lane == base + 3, A, 0.0), axis=1, keepdims=True)
    x1c, y1c, x2c, y2c = _decode(rx, ry, rw, rh,
                                 P[:, 0:1], P[:, 1:2], P[:, 2:3], P[:, 3:4])
    off = icls.astype(jnp.float32) * _OFF
    ax1, ay1, ax2, ay2 = x1c + off, y1c + off, x2c + off, y2c + off
    sc = sc_ref[...]
    packT = jnp.concatenate(
        [ax1, ay1, ax2, ay2, sc, jnp.zeros((_KP, 3), jnp.float32)], axis=1).T
    bx1, by1, bx2, by2 = packT[0:1, :], packT[1:2, :], packT[2:3, :], packT[3:4, :]
    areac = (ax2 - ax1) * (ay2 - ay1)
    arear = (bx2 - bx1) * (by2 - by1)
    iw = jnp.maximum(jnp.minimum(ax2, bx2) - jnp.maximum(ax1, bx1), 0.0)
    ih = jnp.maximum(jnp.minimum(ay2, by2) - jnp.maximum(ay1, by1), 0.0)
    inter = iw * ih
    iou_ref[...] = inter / (areac + arear - inter + 1e-9)
    scr_ref[:, 0:1] = x1c
    scr_ref[:, 1:2] = y1c
    scr_ref[:, 2:3] = x2c
    scr_ref[:, 3:4] = y2c
    scr_ref[:, 4:5] = sc
    scr_ref[:, 5:8] = jnp.zeros((_KP, 3), jnp.float32)
    sw_ref[...] = packT[4:5, :]
    lane_r = jax.lax.broadcasted_iota(jnp.int32, (1, _KP), 1)

    def body(i, carry):
        sw = sw_ref[...]
        m = jnp.max(sw)
        j = jnp.min(jnp.where(sw == m, lane_r, _KP))
        out_ref[pl.ds(i, 1), 0:5] = scr_ref[pl.ds(j, 1), 0:5]
        row = iou_ref[pl.ds(j, 1), :]
        sw_ref[...] = jnp.where(row > _NMS_THRESH, -jnp.inf, sw)
        return carry

    jax.lax.fori_loop(0, _DET, body, 0)


@jax.jit
def kernel(class_logits, box_regression, proposals):
    scores_m = pl.pallas_call(
        _scores_body,
        out_shape=jax.ShapeDtypeStruct((_N, _C), jnp.float32),
    )(class_logits)

    top_scores, top_idx = jax.lax.top_k(scores_m.reshape(-1), _K)
    rows = top_idx // _C
    icls = top_idx % _C

    pad = _KP - _K
    ridp = jnp.concatenate(
        [rows, jnp.full((pad,), -1, jnp.int32)], axis=0)[:, None]
    iclsp = jnp.concatenate(
        [icls, jnp.zeros((pad,), jnp.int32)], axis=0)[:, None]
    scp = jnp.concatenate(
        [top_scores, jnp.full((pad,), -jnp.inf, jnp.float32)], axis=0)[:, None]

    bp = jnp.concatenate([box_regression, proposals], axis=1)
    out = pl.pallas_call(
        _nms_body,
        out_shape=jax.ShapeDtypeStruct((_DET, 8), jnp.float32),
        scratch_shapes=[
            pltpu.VMEM((_KP, 8), jnp.float32),
            pltpu.VMEM((_KP, _KP), jnp.float32),
            pltpu.VMEM((1, _KP), jnp.float32),
        ],
    )(bp, ridp, iclsp, scp)
    return out[:, :5]


# speedup vs baseline: 2.9469x; 2.0283x over previous
import jax
import jax.numpy as jnp
import numpy as np
from jax.experimental import pallas as pl
from jax.experimental.pallas import tpu as pltpu

_N = 5000
_C = 91
_TOP_ROW = 20
_SCORE_THRESH = 0.05
_NMS_THRESH = 0.5
_DET = 100
_K = 1000
_KP = 1024
_IMG = 800.0
_OFF = _IMG + 1.0
_CLIP = float(np.log(1000.0 / 16.0))


def _scores_body(logits_ref, vals_ref, fidx_ref):
    x = logits_ref[...]
    m = jnp.max(x, axis=-1, keepdims=True)
    e = jnp.exp(x - m)
    p = e / jnp.sum(e, axis=-1, keepdims=True)
    cls = jax.lax.broadcasted_iota(jnp.int32, x.shape, 1)
    valid = (cls >= 1) & (p > _SCORE_THRESH)
    cur = jnp.where(valid, p, -1.0)
    row91 = jax.lax.broadcasted_iota(jnp.int32, x.shape, 0) * _C
    for s in range(_TOP_ROW):
        mv = jnp.max(cur, axis=1, keepdims=True)
        ji = jnp.min(jnp.where(cur == mv, cls, _C), axis=1,
                     keepdims=True)
        vals_ref[:, s:s + 1] = mv
        fidx_ref[:, s:s + 1] = (ji + row91[:, 0:1]).astype(jnp.float32)
        cur = jnp.where(cls == ji, -2.0, cur)


def _decode(rx, ry, rw, rh, px1, py1, px2, py2):
    w = px2 - px1
    h = py2 - py1
    cx = px1 + 0.5 * w
    cy = py1 + 0.5 * h
    dx = rx / 10.0
    dy = ry / 10.0
    dw = jnp.minimum(rw / 5.0, _CLIP)
    dh = jnp.minimum(rh / 5.0, _CLIP)
    pcx = dx * w + cx
    pcy = dy * h + cy
    pw = jnp.exp(dw) * w
    ph = jnp.exp(dh) * h
    x1 = jnp.clip(pcx - 0.5 * pw, 0.0, _IMG)
    y1 = jnp.clip(pcy - 0.5 * ph, 0.0, _IMG)
    x2 = jnp.clip(pcx + 0.5 * pw, 0.0, _IMG)
    y2 = jnp.clip(pcy + 0.5 * ph, 0.0, _IMG)
    return x1, y1, x2, y2


def _nms_body(breg_ref, prop_ref, fidx20_ref, rid_ref, slot_ref, sc_ref,
              out_ref, scr_ref, iou_ref, sw_ref):
    rid = rid_ref[...]
    dn = (((1,), (0,)), ((), ()))
    nb = 5
    bs = _N // nb

    def gather_step(b, accs):
        accA, accP, accX = accs
        li = jax.lax.broadcasted_iota(jnp.int32, (_KP, bs), 1) + b * bs
        ohb = (li == rid).astype(jnp.float32)
        off = pl.multiple_of(b * bs, 8)
        accA = accA + jax.lax.dot_general(
            ohb, breg_ref[pl.ds(off, bs), :], dn,
            precision=jax.lax.Precision.HIGHEST,
            preferred_element_type=jnp.float32)
        accP = accP + jax.lax.dot_general(
            ohb, prop_ref[pl.ds(off, bs), :], dn,
            precision=jax.lax.Precision.HIGHEST,
            preferred_element_type=jnp.float32)
        accX = accX + jax.lax.dot_general(
            ohb, fidx20_ref[pl.ds(off, bs), :], dn,
            precision=jax.lax.Precision.HIGHEST,
            preferred_element_type=jnp.float32)
        return accA, accP, accX

    A, P, X = jax.lax.fori_loop(
        0, nb, gather_step,
        (jnp.zeros((_KP, 4 * _C), jnp.float32),
         jnp.zeros((_KP, 4), jnp.float32),
         jnp.zeros((_KP, _TOP_ROW), jnp.float32)))
    slot = slot_ref[...]
    lane20 = jax.lax.broadcasted_iota(jnp.int32, (_KP, _TOP_ROW), 1)
    fidx = jnp.sum(jnp.where(lane20 == slot, X, 0.0), axis=1,
                   keepdims=True).astype(jnp.int32)
    icls = fidx - rid * _C
    lane = jax.lax.broadcasted_iota(jnp.int32, (_KP, 4 * _C), 1)
    base = icls * 4
    rx = jnp.sum(jnp.where(lane == base, A, 0.0), axis=1, keepdims=True)
    ry = jnp.sum(jnp.where(lane == base + 1, A, 0.0), axis=1, keepdims=True)
    rw = jnp.sum(jnp.where(lane == base + 2, A, 0.0), axis=1, keepdims=True)
    rh = jnp.sum(jnp.where(lane == base + 3, A, 0.0), axis=1, keepdims=True)
    x1c, y1c, x2c, y2c = _decode(rx, ry, rw, rh,
                                 P[:, 0:1], P[:, 1:2], P[:, 2:3], P[:, 3:4])
    off = icls.astype(jnp.float32) * _OFF
    ax1, ay1, ax2, ay2 = x1c + off, y1c + off, x2c + off, y2c + off
    sc = sc_ref[...]
    packT = jnp.concatenate(
        [ax1, ay1, ax2, ay2, sc, jnp.zeros((_KP, 3), jnp.float32)], axis=1).T
    bx1, by1, bx2, by2 = packT[0:1, :], packT[1:2, :], packT[2:3, :], packT[3:4, :]
    areac = (ax2 - ax1) * (ay2 - ay1)
    arear = (bx2 - bx1) * (by2 - by1)
    iw = jnp.maximum(jnp.minimum(ax2, bx2) - jnp.maximum(ax1, bx1), 0.0)
    ih = jnp.maximum(jnp.minimum(ay2, by2) - jnp.maximum(ay1, by1), 0.0)
    inter = iw * ih
    iou_ref[...] = inter / (areac + arear - inter + 1e-9)
    scr_ref[:, 0:1] = x1c
    scr_ref[:, 1:2] = y1c
    scr_ref[:, 2:3] = x2c
    scr_ref[:, 3:4] = y2c
    scr_ref[:, 4:5] = sc
    scr_ref[:, 5:8] = jnp.zeros((_KP, 3), jnp.float32)
    sw_ref[...] = packT[4:5, :]
    lane_r = jax.lax.broadcasted_iota(jnp.int32, (1, _KP), 1)

    def body(i, carry):
        sw = sw_ref[...]
        m = jnp.max(sw)
        j = jnp.min(jnp.where(sw == m, lane_r, _KP))
        out_ref[pl.ds(i, 1), 0:5] = scr_ref[pl.ds(j, 1), 0:5]
        row = iou_ref[pl.ds(j, 1), :]
        sw_ref[...] = jnp.where(row > _NMS_THRESH, -jnp.inf, sw)
        return carry

    jax.lax.fori_loop(0, _DET, body, 0)


@jax.jit
def kernel(class_logits, box_regression, proposals):
    vals, fidx20 = pl.pallas_call(
        _scores_body,
        out_shape=(jax.ShapeDtypeStruct((_N, _TOP_ROW), jnp.float32),
                   jax.ShapeDtypeStruct((_N, _TOP_ROW), jnp.float32)),
    )(class_logits)

    top_scores, p = jax.lax.top_k(vals.reshape(-1), _K)
    rows = p // _TOP_ROW
    slots = p % _TOP_ROW

    pad = _KP - _K
    ridp = jnp.concatenate(
        [rows, jnp.full((pad,), -1, jnp.int32)], axis=0)[:, None]
    slotp = jnp.concatenate(
        [slots, jnp.zeros((pad,), jnp.int32)], axis=0)[:, None]
    scp = jnp.concatenate(
        [top_scores, jnp.full((pad,), -jnp.inf, jnp.float32)], axis=0)[:, None]

    out = pl.pallas_call(
        _nms_body,
        out_shape=jax.ShapeDtypeStruct((_DET, 8), jnp.float32),
        scratch_shapes=[
            pltpu.VMEM((_KP, 8), jnp.float32),
            pltpu.VMEM((_KP, _KP), jnp.float32),
            pltpu.VMEM((1, _KP), jnp.float32),
        ],
    )(box_regression, proposals, fidx20, ridp, slotp, scp)
    return out[:, :5]


# speedup vs baseline: 3.8538x; 1.3077x over previous
import jax
import jax.numpy as jnp
import numpy as np
from jax.experimental import pallas as pl
from jax.experimental.pallas import tpu as pltpu

_N = 5000
_C = 91
_TOP_ROW = 20
_SCORE_THRESH = 0.05
_NMS_THRESH = 0.5
_DET = 100
_K = 1000
_KP = 1024
_IMG = 800.0
_OFF = _IMG + 1.0
_CLIP = float(np.log(1000.0 / 16.0))


def _scores_body(logits_ref, vals_ref, fidx_ref):
    x = logits_ref[...]
    m = jnp.max(x, axis=-1, keepdims=True)
    e = jnp.exp(x - m)
    p = e / jnp.sum(e, axis=-1, keepdims=True)
    cls = jax.lax.broadcasted_iota(jnp.int32, x.shape, 1)
    valid = (cls >= 1) & (p > _SCORE_THRESH)
    cur = jnp.where(valid, p, -1.0)
    row91 = jax.lax.broadcasted_iota(jnp.int32, x.shape, 0) * _C
    for s in range(_TOP_ROW):
        mv = jnp.max(cur, axis=1, keepdims=True)
        ji = jnp.min(jnp.where(cur == mv, cls, _C), axis=1,
                     keepdims=True)
        vals_ref[:, s:s + 1] = mv
        fidx_ref[:, s:s + 1] = (ji + row91[:, 0:1]).astype(jnp.float32)
        cur = jnp.where(cls == ji, -2.0, cur)


def _decode(rx, ry, rw, rh, px1, py1, px2, py2):
    w = px2 - px1
    h = py2 - py1
    cx = px1 + 0.5 * w
    cy = py1 + 0.5 * h
    dx = rx / 10.0
    dy = ry / 10.0
    dw = jnp.minimum(rw / 5.0, _CLIP)
    dh = jnp.minimum(rh / 5.0, _CLIP)
    pcx = dx * w + cx
    pcy = dy * h + cy
    pw = jnp.exp(dw) * w
    ph = jnp.exp(dh) * h
    x1 = jnp.clip(pcx - 0.5 * pw, 0.0, _IMG)
    y1 = jnp.clip(pcy - 0.5 * ph, 0.0, _IMG)
    x2 = jnp.clip(pcx + 0.5 * pw, 0.0, _IMG)
    y2 = jnp.clip(pcy + 0.5 * ph, 0.0, _IMG)
    return x1, y1, x2, y2


def _nms_body(breg_ref, prop_ref, fidx20_ref, rid_ref, slot_ref, sc_ref,
              out_ref, scr_ref, iou_ref, sw_ref):
    rid = rid_ref[...]
    dn = (((1,), (0,)), ((), ()))
    nb = 5
    bs = _N // nb

    def gather_step(b, accs):
        accA, accP, accX = accs
        li = jax.lax.broadcasted_iota(jnp.int32, (_KP, bs), 1) + b * bs
        ohb = (li == rid).astype(jnp.float32)
        off = pl.multiple_of(b * bs, 8)
        accA = accA + jax.lax.dot_general(
            ohb, breg_ref[pl.ds(off, bs), :], dn,
            precision=jax.lax.Precision.HIGHEST,
            preferred_element_type=jnp.float32)
        accP = accP + jax.lax.dot_general(
            ohb, prop_ref[pl.ds(off, bs), :], dn,
            precision=jax.lax.Precision.HIGHEST,
            preferred_element_type=jnp.float32)
        accX = accX + jax.lax.dot_general(
            ohb, fidx20_ref[pl.ds(off, bs), :], dn,
            precision=jax.lax.Precision.HIGHEST,
            preferred_element_type=jnp.float32)
        return accA, accP, accX

    A, P, X = jax.lax.fori_loop(
        0, nb, gather_step,
        (jnp.zeros((_KP, 4 * _C), jnp.float32),
         jnp.zeros((_KP, 4), jnp.float32),
         jnp.zeros((_KP, _TOP_ROW), jnp.float32)))
    slot = slot_ref[...]
    lane20 = jax.lax.broadcasted_iota(jnp.int32, (_KP, _TOP_ROW), 1)
    fidx = jnp.sum(jnp.where(lane20 == slot, X, 0.0), axis=1,
                   keepdims=True).astype(jnp.int32)
    icls = fidx - rid * _C
    lane = jax.lax.broadcasted_iota(jnp.int32, (_KP, 4 * _C), 1)
    base = icls * 4
    rx = jnp.sum(jnp.where(lane == base, A, 0.0), axis=1, keepdims=True)
    ry = jnp.sum(jnp.where(lane == base + 1, A, 0.0), axis=1, keepdims=True)
    rw = jnp.sum(jnp.where(lane == base + 2, A, 0.0), axis=1, keepdims=True)
    rh = jnp.sum(jnp.where(lane == base + 3, A, 0.0), axis=1, keepdims=True)
    x1c, y1c, x2c, y2c = _decode(rx, ry, rw, rh,
                                 P[:, 0:1], P[:, 1:2], P[:, 2:3], P[:, 3:4])
    off = icls.astype(jnp.float32) * _OFF
    ax1, ay1, ax2, ay2 = x1c + off, y1c + off, x2c + off, y2c + off
    sc = sc_ref[...]
    packT = jnp.concatenate(
        [ax1, ay1, ax2, ay2, sc, jnp.zeros((_KP, 3), jnp.float32)], axis=1).T
    bx1, by1, bx2, by2 = packT[0:1, :], packT[1:2, :], packT[2:3, :], packT[3:4, :]
    areac = (ax2 - ax1) * (ay2 - ay1)
    arear = (bx2 - bx1) * (by2 - by1)
    iw = jnp.maximum(jnp.minimum(ax2, bx2) - jnp.maximum(ax1, bx1), 0.0)
    ih = jnp.maximum(jnp.minimum(ay2, by2) - jnp.maximum(ay1, by1), 0.0)
    inter = iw * ih
    iou_ref[...] = inter / (areac + arear - inter + 1e-9)
    scr_ref[:, 0:1] = x1c
    scr_ref[:, 1:2] = y1c
    scr_ref[:, 2:3] = x2c
    scr_ref[:, 3:4] = y2c
    scr_ref[:, 4:5] = sc
    scr_ref[:, 5:8] = jnp.zeros((_KP, 3), jnp.float32)
    sw_ref[...] = packT[4:5, :]
    lane_r = jax.lax.broadcasted_iota(jnp.int32, (1, _KP), 1)

    def body(i, carry):
        sw = sw_ref[...]
        m = jnp.max(sw)
        j = jnp.min(jnp.where(sw == m, lane_r, _KP))
        out_ref[pl.ds(i, 1), 0:5] = scr_ref[pl.ds(j, 1), 0:5]
        row = iou_ref[pl.ds(j, 1), :]
        sw_ref[...] = jnp.where(row > _NMS_THRESH, -jnp.inf, sw)
        return carry

    jax.lax.fori_loop(0, _DET, body, 0)


@jax.jit
def kernel(class_logits, box_regression, proposals):
    vals, fidx20 = pl.pallas_call(
        _scores_body,
        out_shape=(jax.ShapeDtypeStruct((_N, _TOP_ROW), jnp.float32),
                   jax.ShapeDtypeStruct((_N, _TOP_ROW), jnp.float32)),
    )(class_logits)

    top_scores, p = jax.lax.approx_max_k(
        vals.reshape(-1), _K, recall_target=1.0)
    rows = p // _TOP_ROW
    slots = p % _TOP_ROW

    pad = _KP - _K
    ridp = jnp.concatenate(
        [rows, jnp.full((pad,), -1, jnp.int32)], axis=0)[:, None]
    slotp = jnp.concatenate(
        [slots, jnp.zeros((pad,), jnp.int32)], axis=0)[:, None]
    scp = jnp.concatenate(
        [top_scores, jnp.full((pad,), -jnp.inf, jnp.float32)], axis=0)[:, None]

    out = pl.pallas_call(
        _nms_body,
        out_shape=jax.ShapeDtypeStruct((_DET, 8), jnp.float32),
        scratch_shapes=[
            pltpu.VMEM((_KP, 8), jnp.float32),
            pltpu.VMEM((_KP, _KP), jnp.float32),
            pltpu.VMEM((1, _KP), jnp.float32),
        ],
    )(box_regression, proposals, fidx20, ridp, slotp, scp)
    return out[:, :5]
